# Initial kernel scaffold; baseline (speedup 1.0000x reference)
import jax
import jax.numpy as jnp
from jax import lax
from jax.experimental import pallas as pl
from jax.experimental.pallas import tpu as pltpu
from jax.experimental.pallas import tpu_sc as plsc

E = 160000
T = 320000
F = 128

NC = 2
NS = 16
L = 16


def _premlp_body(x_ref, w_ref, b_ref, o_ref):
    o = jnp.dot(x_ref[...], w_ref[...], preferred_element_type=jnp.float32)
    o = o + b_ref[...]
    o_ref[...] = o * jax.nn.sigmoid(o)


def _premlp(m, W_m, b_m):
    BE = 1280
    return pl.pallas_call(
        _premlp_body,
        grid=(E // BE,),
        in_specs=[
            pl.BlockSpec((BE, F), lambda i: (i, 0)),
            pl.BlockSpec((F, F), lambda i: (0, 0)),
            pl.BlockSpec((1, F), lambda i: (0, 0)),
        ],
        out_specs=pl.BlockSpec((BE, F), lambda i: (i, 0)),
        out_shape=jax.ShapeDtypeStruct((E, F), jnp.float32),
    )(m, W_m, b_m.reshape(1, F))


def _bilinear_body(g_ref, sbf_ref, wsbf_ref, bsbf_ref, wt_ref, bbil_ref, o_ref):
    s = jnp.dot(sbf_ref[...], wsbf_ref[...], preferred_element_type=jnp.float32)
    s = s + bsbf_ref[...]
    s = s * jax.nn.sigmoid(s)
    g = g_ref[...]
    acc = jnp.broadcast_to(bbil_ref[...], g.shape)
    for i in range(8):
        acc = acc + s[:, i:i + 1] * jnp.dot(
            g, wt_ref[i], preferred_element_type=jnp.float32)
    o_ref[...] = acc


def _bilinear(gathered, sbf, W_sbf, b_sbf, Wt, b_bil):
    BT = 1280
    S = sbf.shape[1]
    return pl.pallas_call(
        _bilinear_body,
        grid=(T // BT,),
        in_specs=[
            pl.BlockSpec((BT, F), lambda i: (i, 0)),
            pl.BlockSpec((BT, S), lambda i: (i, 0)),
            pl.BlockSpec((S, 8), lambda i: (0, 0)),
            pl.BlockSpec((1, 8), lambda i: (0, 0)),
            pl.BlockSpec((8, F, F), lambda i: (0, 0, 0)),
            pl.BlockSpec((1, F), lambda i: (0, 0)),
        ],
        out_specs=pl.BlockSpec((BT, F), lambda i: (i, 0)),
        out_shape=jax.ShapeDtypeStruct((T, F), jnp.float32),
    )(gathered, sbf, W_sbf, b_sbf.reshape(1, 8), Wt, b_bil.reshape(1, F))


def _tail_body(agg_ref, m_ref, wd_ref, bd_ref, w1a_ref, b1a_ref, w1b_ref,
               b1b_ref, w2a_ref, b2a_ref, w2b_ref, b2b_ref, o_ref):
    def sw(x):
        return x * jax.nn.sigmoid(x)

    def mm(x, w):
        return jnp.dot(x, w, preferred_element_type=jnp.float32)

    u = sw(mm(agg_ref[...], wd_ref[...]) + bd_ref[...])
    m1 = m_ref[...] + u
    m2 = m1 + (mm(sw(mm(m1, w1a_ref[...]) + b1a_ref[...]), w1b_ref[...])
               + b1b_ref[...])
    m3 = m2 + (mm(sw(mm(m2, w2a_ref[...]) + b2a_ref[...]), w2b_ref[...])
               + b2b_ref[...])
    o_ref[...] = m3


def _tail(agg, m, W_down, b_down, W1a, b1a, W1b, b1b, W2a, b2a, W2b, b2b):
    BE = 1280
    wspec = pl.BlockSpec((F, F), lambda i: (0, 0))
    bspec = pl.BlockSpec((1, F), lambda i: (0, 0))
    return pl.pallas_call(
        _tail_body,
        grid=(E // BE,),
        in_specs=[
            pl.BlockSpec((BE, F), lambda i: (i, 0)),
            pl.BlockSpec((BE, F), lambda i: (i, 0)),
            wspec, bspec, wspec, bspec, wspec, bspec, wspec, bspec, wspec,
            bspec,
        ],
        out_specs=pl.BlockSpec((BE, F), lambda i: (i, 0)),
        out_shape=jax.ShapeDtypeStruct((E, F), jnp.float32),
    )(agg, m, W_down, b_down.reshape(1, F), W1a, b1a.reshape(1, F), W1b,
      b1b.reshape(1, F), W2a, b2a.reshape(1, F), W2b, b2b.reshape(1, F))


_G_CH = 400
_G_TW = T // (NC * NS)


def _gather_kernel(table_hbm, idx_hbm, out_hbm, idx_c, rows_v, sem):
    wid = lax.axis_index("s") * NC + lax.axis_index("c")
    base = pl.multiple_of(wid * _G_TW, 8)
    for g in range(_G_TW // _G_CH):
        off = pl.multiple_of(base + g * _G_CH, 8)
        pltpu.sync_copy(idx_hbm.at[pl.ds(off, _G_CH)], idx_c)
        pltpu.async_copy(table_hbm.at[idx_c], rows_v, sem).wait()
        pltpu.sync_copy(rows_v, out_hbm.at[pl.ds(off, _G_CH)])


def _gather(m_kj, idx_kj):
    mesh = plsc.VectorSubcoreMesh(core_axis_name="c", subcore_axis_name="s")
    return pl.kernel(
        _gather_kernel,
        out_type=jax.ShapeDtypeStruct((T, F), jnp.float32),
        mesh=mesh,
        scratch_types=[
            pltpu.VMEM((_G_CH,), jnp.int32),
            pltpu.VMEM((_G_CH, F), jnp.float32),
            pltpu.SemaphoreType.DMA,
        ],
    )(m_kj, idx_kj)


_S_EC = E // NC
_S_NP = 5
_S_R = _S_EC // _S_NP
_S_TPW = T // NS
_S_CH = 128
_S_NROW = (_S_TPW + 2 * _S_CH - 1) // _S_CH + 1


def _scatter_kernel(bil_hbm, idxji_hbm, zeros_hbm, out_hbm, jbuf, tids, lids,
                    rows_v, acc, sem):
    c = lax.axis_index("c")
    s = lax.axis_index("s")
    iota16 = lax.iota(jnp.int32, L)

    pltpu.sync_copy(idxji_hbm.at[pl.ds(pl.multiple_of(s * _S_TPW, 8), _S_TPW)],
                    jbuf)

    for p in range(_S_NP):
        base = c * _S_EC + p * _S_R

        zoff = pl.multiple_of(s * (_S_R // NS), 8)
        pltpu.sync_copy(zeros_hbm, acc.at[pl.ds(zoff, _S_R // NS)])
        plsc.subcore_barrier()

        def scan_body(k, cnt):
            v = jbuf[pl.ds(pl.multiple_of(k * L, 8), L)]
            rel = v - base
            msk = (rel >= 0) & (rel < _S_R)
            mi = msk.astype(jnp.int32)
            pos = (cnt + plsc.cumsum(mi)) - mi
            prow = lax.shift_right_logical(pos, 7)
            pcol = lax.bitwise_and(pos, _S_CH - 1)
            tid = (s * _S_TPW + k * L) + iota16
            plsc.store_scatter(tids, [prow, pcol], tid, msk)
            plsc.store_scatter(lids, [prow, pcol], rel, msk)
            return cnt + jnp.sum(mi)

        cnt = lax.fori_loop(0, _S_TPW // L, scan_body, jnp.int32(0))

        for j in range(_S_CH // L):
            pos = cnt + j * L + iota16
            prow = lax.shift_right_logical(pos, 7)
            pcol = lax.bitwise_and(pos, _S_CH - 1)
            tidpad = (s * _S_TPW + j * L) + iota16
            lidpad = jnp.full((L,), _S_R, jnp.int32)
            plsc.store_scatter(tids, [prow, pcol], tidpad, None)
            plsc.store_scatter(lids, [prow, pcol], lidpad, None)

        nch = lax.div(cnt + (_S_CH - 1), jnp.int32(_S_CH))

        def gs_body(ch, carry):
            pltpu.async_copy(bil_hbm.at[tids.at[ch]], rows_v, sem).wait()
            pltpu.sync_copy(rows_v, acc.at[lids.at[ch]], add=True)
            return carry

        lax.fori_loop(0, nch, gs_body, jnp.int32(0))
        plsc.subcore_barrier()

        woff = pl.multiple_of(base + s * (_S_R // NS), 8)
        pltpu.sync_copy(acc.at[pl.ds(zoff, _S_R // NS)],
                        out_hbm.at[pl.ds(woff, _S_R // NS)])
        plsc.subcore_barrier()


def _scatter(bil, idx_ji):
    mesh = plsc.VectorSubcoreMesh(core_axis_name="c", subcore_axis_name="s")
    zeros = jnp.zeros((_S_R // NS, F), jnp.float32)
    return pl.kernel(
        _scatter_kernel,
        out_type=jax.ShapeDtypeStruct((E, F), jnp.float32),
        mesh=mesh,
        scratch_types=[
            pltpu.VMEM((_S_TPW,), jnp.int32),
            pltpu.VMEM((_S_NROW, _S_CH), jnp.int32),
            pltpu.VMEM((_S_NROW, _S_CH), jnp.int32),
            pltpu.VMEM((_S_CH, F), jnp.float32),
            pltpu.VMEM_SHARED((_S_R + 8, F), jnp.float32),
            pltpu.SemaphoreType.DMA,
        ],
    )(bil, idx_ji, zeros)


def kernel(m, rbf, sbf, idx_kj, idx_ji, W_rbf, W_sbf, b_sbf, W_m, b_m, W_bil,
           b_bil, W_down, b_down, W1a, b1a, W1b, b1b, W2a, b2a, W2b, b2b):
    del rbf, W_rbf
    m_kj = _premlp(m, W_m, b_m)
    gathered = _gather(m_kj, idx_kj)
    Wt = jnp.transpose(W_bil, (1, 2, 0))
    bil = _bilinear(gathered, sbf, W_sbf, b_sbf, Wt, b_bil)
    agg = _scatter(bil, idx_ji)
    return _tail(agg, m, W_down, b_down, W1a, b1a, W1b, b1b, W2a, b2a, W2b,
                 b2b)


# speedup vs baseline: 1.7901x; 1.7901x over previous
import jax
import jax.numpy as jnp
from jax import lax
from jax.experimental import pallas as pl
from jax.experimental.pallas import tpu as pltpu
from jax.experimental.pallas import tpu_sc as plsc

E = 160000
T = 320000
F = 128

NC = 2
NS = 16
L = 16


def _premlp_body(x_ref, w_ref, b_ref, o_ref):
    o = jnp.dot(x_ref[...], w_ref[...], preferred_element_type=jnp.float32)
    o = o + b_ref[...]
    o_ref[...] = o * jax.nn.sigmoid(o)


def _premlp(m, W_m, b_m):
    BE = 1280
    return pl.pallas_call(
        _premlp_body,
        grid=(E // BE,),
        in_specs=[
            pl.BlockSpec((BE, F), lambda i: (i, 0)),
            pl.BlockSpec((F, F), lambda i: (0, 0)),
            pl.BlockSpec((1, F), lambda i: (0, 0)),
        ],
        out_specs=pl.BlockSpec((BE, F), lambda i: (i, 0)),
        out_shape=jax.ShapeDtypeStruct((E, F), jnp.float32),
    )(m, W_m, b_m.reshape(1, F))


def _bilinear_body(g_ref, sbf_ref, wsbf_ref, bsbf_ref, wt_ref, bbil_ref, o_ref):
    s = jnp.dot(sbf_ref[...], wsbf_ref[...], preferred_element_type=jnp.float32)
    s = s + bsbf_ref[...]
    s = s * jax.nn.sigmoid(s)
    g = g_ref[...]
    acc = jnp.broadcast_to(bbil_ref[...], g.shape)
    for i in range(8):
        acc = acc + s[:, i:i + 1] * jnp.dot(
            g, wt_ref[i], preferred_element_type=jnp.float32)
    o_ref[...] = acc


def _bilinear(gathered, sbf, W_sbf, b_sbf, Wt, b_bil):
    BT = 1280
    S = sbf.shape[1]
    return pl.pallas_call(
        _bilinear_body,
        grid=(T // BT,),
        in_specs=[
            pl.BlockSpec((BT, F), lambda i: (i, 0)),
            pl.BlockSpec((BT, S), lambda i: (i, 0)),
            pl.BlockSpec((S, 8), lambda i: (0, 0)),
            pl.BlockSpec((1, 8), lambda i: (0, 0)),
            pl.BlockSpec((8, F, F), lambda i: (0, 0, 0)),
            pl.BlockSpec((1, F), lambda i: (0, 0)),
        ],
        out_specs=pl.BlockSpec((BT, F), lambda i: (i, 0)),
        out_shape=jax.ShapeDtypeStruct((T, F), jnp.float32),
    )(gathered, sbf, W_sbf, b_sbf.reshape(1, 8), Wt, b_bil.reshape(1, F))


def _tail_body(agg_ref, m_ref, wd_ref, bd_ref, w1a_ref, b1a_ref, w1b_ref,
               b1b_ref, w2a_ref, b2a_ref, w2b_ref, b2b_ref, o_ref):
    def sw(x):
        return x * jax.nn.sigmoid(x)

    def mm(x, w):
        return jnp.dot(x, w, preferred_element_type=jnp.float32)

    u = sw(mm(agg_ref[...], wd_ref[...]) + bd_ref[...])
    m1 = m_ref[...] + u
    m2 = m1 + (mm(sw(mm(m1, w1a_ref[...]) + b1a_ref[...]), w1b_ref[...])
               + b1b_ref[...])
    m3 = m2 + (mm(sw(mm(m2, w2a_ref[...]) + b2a_ref[...]), w2b_ref[...])
               + b2b_ref[...])
    o_ref[...] = m3


def _tail(agg, m, W_down, b_down, W1a, b1a, W1b, b1b, W2a, b2a, W2b, b2b):
    BE = 1280
    wspec = pl.BlockSpec((F, F), lambda i: (0, 0))
    bspec = pl.BlockSpec((1, F), lambda i: (0, 0))
    return pl.pallas_call(
        _tail_body,
        grid=(E // BE,),
        in_specs=[
            pl.BlockSpec((BE, F), lambda i: (i, 0)),
            pl.BlockSpec((BE, F), lambda i: (i, 0)),
            wspec, bspec, wspec, bspec, wspec, bspec, wspec, bspec, wspec,
            bspec,
        ],
        out_specs=pl.BlockSpec((BE, F), lambda i: (i, 0)),
        out_shape=jax.ShapeDtypeStruct((E, F), jnp.float32),
    )(agg, m, W_down, b_down.reshape(1, F), W1a, b1a.reshape(1, F), W1b,
      b1b.reshape(1, F), W2a, b2a.reshape(1, F), W2b, b2b.reshape(1, F))


_G_CH = 400
_G_TW = T // (NC * NS)


def _gather_kernel(table_hbm, idx_hbm, out_hbm, idx_c, rows_v, sem):
    wid = lax.axis_index("s") * NC + lax.axis_index("c")
    base = pl.multiple_of(wid * _G_TW, 8)
    for g in range(_G_TW // _G_CH):
        off = pl.multiple_of(base + g * _G_CH, 8)
        pltpu.sync_copy(idx_hbm.at[pl.ds(off, _G_CH)], idx_c)
        pltpu.async_copy(table_hbm.at[idx_c], rows_v, sem).wait()
        pltpu.sync_copy(rows_v, out_hbm.at[pl.ds(off, _G_CH)])


def _gather(m_kj, idx_kj):
    mesh = plsc.VectorSubcoreMesh(core_axis_name="c", subcore_axis_name="s")
    return pl.kernel(
        _gather_kernel,
        out_type=jax.ShapeDtypeStruct((T, F), jnp.float32),
        mesh=mesh,
        scratch_types=[
            pltpu.VMEM((_G_CH,), jnp.int32),
            pltpu.VMEM((_G_CH, F), jnp.float32),
            pltpu.SemaphoreType.DMA,
        ],
    )(m_kj, idx_kj)


_S_EC = E // NC
_S_PS = [13312, 13312, 13312, 13312, 13312, 13440]
_S_RMAX = 13440
_S_DUMP = 64
_S_CH = 80
_S_CPT = T // _S_CH // NS


def _scatter_kernel(bil_hbm, idxji_hbm, zeros_hbm, out_hbm, jc, selbuf,
                    rows_v, acc, sg0, sg1, si0, si1):
    c = lax.axis_index("c")
    s = lax.axis_index("s")
    iota16 = lax.iota(jnp.int32, L)
    cs = s * _S_CPT
    sgs = (sg0, sg1)
    sis = (si0, si1)

    def g_copy(ch, par):
        off = pl.multiple_of((cs + ch) * _S_CH, 8)
        return pltpu.make_async_copy(bil_hbm.at[pl.ds(off, _S_CH)],
                                     rows_v.at[par], sgs[par])

    def i_copy(ch, par):
        off = pl.multiple_of((cs + ch) * _S_CH, 8)
        return pltpu.make_async_copy(idxji_hbm.at[pl.ds(off, _S_CH)],
                                     jc.at[par], sis[par])

    off_p = 0
    for rp in _S_PS:
        base = c * _S_EC + off_p
        off_p += rp

        z0 = pl.multiple_of(s * (_S_RMAX // NS), 8)
        pltpu.sync_copy(zeros_hbm, acc.at[pl.ds(z0, _S_RMAX // NS)])
        plsc.subcore_barrier()

        g_copy(0, 0).start()
        i_copy(0, 0).start()

        def pair_body(ch2, carry):
            for par in range(2):
                ch = ch2 * 2 + par
                g_copy(ch, par).wait()
                i_copy(ch, par).wait()
                if par == 0:
                    g_copy(ch + 1, 1).start()
                    i_copy(ch + 1, 1).start()
                else:
                    @pl.when(ch2 < _S_CPT // 2 - 1)
                    def _():
                        g_copy(ch + 1, 0).start()
                        i_copy(ch + 1, 0).start()
                for j in range(_S_CH // L):
                    ko = ch * _S_CH + j * L
                    v = jc[par, pl.ds(j * L, L)]
                    rel = v - base
                    inr = (rel >= 0) & (rel < rp)
                    dump = _S_RMAX + lax.bitwise_and(ko + iota16, _S_DUMP - 1)
                    selbuf[0, pl.ds(j * L, L)] = jnp.where(inr, rel, dump)
                pltpu.sync_copy(rows_v.at[par], acc.at[selbuf.at[0]],
                                add=True)
            return carry

        lax.fori_loop(0, _S_CPT // 2, pair_body, jnp.int32(0))
        plsc.subcore_barrier()

        rpt = rp // NS
        woff = pl.multiple_of(base + s * rpt, 8)
        pltpu.sync_copy(acc.at[pl.ds(pl.multiple_of(s * rpt, 8), rpt)],
                        out_hbm.at[pl.ds(woff, rpt)])
        plsc.subcore_barrier()


def _scatter(bil, idx_ji):
    mesh = plsc.VectorSubcoreMesh(core_axis_name="c", subcore_axis_name="s")
    zeros = jnp.zeros((_S_RMAX // NS, F), jnp.float32)
    return pl.kernel(
        _scatter_kernel,
        out_type=jax.ShapeDtypeStruct((E, F), jnp.float32),
        mesh=mesh,
        scratch_types=[
            pltpu.VMEM((2, _S_CH), jnp.int32),
            pltpu.VMEM((1, _S_CH), jnp.int32),
            pltpu.VMEM((2, _S_CH, F), jnp.float32),
            pltpu.VMEM_SHARED((_S_RMAX + _S_DUMP, F), jnp.float32),
            pltpu.SemaphoreType.DMA,
            pltpu.SemaphoreType.DMA,
            pltpu.SemaphoreType.DMA,
            pltpu.SemaphoreType.DMA,
        ],
    )(bil, idx_ji, zeros)


def kernel(m, rbf, sbf, idx_kj, idx_ji, W_rbf, W_sbf, b_sbf, W_m, b_m, W_bil,
           b_bil, W_down, b_down, W1a, b1a, W1b, b1b, W2a, b2a, W2b, b2b):
    del rbf, W_rbf
    m_kj = _premlp(m, W_m, b_m)
    gathered = _gather(m_kj, idx_kj)
    Wt = jnp.transpose(W_bil, (1, 2, 0))
    bil = _bilinear(gathered, sbf, W_sbf, b_sbf, Wt, b_bil)
    agg = _scatter(bil, idx_ji)
    return _tail(agg, m, W_down, b_down, W1a, b1a, W1b, b1b, W2a, b2a, W2b,
                 b2b)
